# Initial kernel scaffold; baseline (speedup 1.0000x reference)
import jax
import jax.numpy as jnp
from jax.experimental import pallas as pl


def kernel(ts_values, table, bin_edges):
    raise NotImplementedError("write your pallas kernel here")


# speedup vs baseline: 86.0318x; 86.0318x over previous
import functools

import jax
import jax.numpy as jnp
from jax import lax
from jax.experimental import pallas as pl
from jax.experimental.pallas import tpu as pltpu
from jax.experimental.pallas import tpu_sc as plsc

_VOCAB = 1024
_HID = 64
_N = 1048576

_EDGE_PAD = 2048

_info = plsc.get_sparse_core_info()
_NC, _NS, _L = _info.num_cores, _info.num_subcores, _info.num_lanes
_NW = _NC * _NS
_PW = _N // _NW
_CH = 128
_NCHUNK = _PW // _CH


def _normalize_body(t_ref, o_ref):
    t = t_ref[...]
    ss = jnp.sum(t * t, axis=1, keepdims=True)
    norm = jnp.sqrt(ss)
    scale = jnp.where(norm > 1.0, 1.0 / norm, jnp.ones_like(norm))
    o_ref[...] = t * scale


def _normalize_table(table):
    return pl.pallas_call(
        _normalize_body,
        out_shape=jax.ShapeDtypeStruct((_VOCAB, _HID), jnp.float32),
    )(table)


def _sc_body(vals_hbm, table_hbm, edges_hbm, emb_out, tok_out,
             edges_v, vals_v, toks_v, rows_v, sem):
    wid = lax.axis_index("s") * _NC + lax.axis_index("c")
    base = wid * _PW

    pltpu.sync_copy(edges_hbm, edges_v)
    pltpu.sync_copy(vals_hbm.at[pl.ds(base, _PW)], vals_v)

    def _search(i, carry):
        v = vals_v[pl.ds(i * _L, _L)]
        pos = jnp.zeros((_L,), jnp.int32)
        k = _VOCAB
        while k >= 1:
            e = plsc.load_gather(edges_v, [pos + (k - 1)])
            pos = jnp.where(e < v, pos + k, pos)
            k //= 2
        tok = jnp.clip(pos - 1, 0, _VOCAB - 1)
        toks_v[pl.ds(i * _L, _L)] = tok
        return carry

    lax.fori_loop(0, _PW // _L, _search, 0)

    pltpu.sync_copy(toks_v, tok_out.at[pl.ds(base, _PW)])

    def _gather(c, carry):
        idx = toks_v.at[pl.ds(c * _CH, _CH)]
        pltpu.async_copy(table_hbm.at[idx], rows_v, sem).wait()
        pltpu.sync_copy(rows_v, emb_out.at[pl.ds(base + c * _CH, _CH)])
        return carry

    lax.fori_loop(0, _NCHUNK, _gather, 0)


_sc_lookup = functools.partial(
    pl.kernel,
    mesh=plsc.VectorSubcoreMesh(core_axis_name="c", subcore_axis_name="s"),
    out_type=[
        jax.ShapeDtypeStruct((_N, _HID), jnp.float32),
        jax.ShapeDtypeStruct((_N,), jnp.int32),
    ],
    scratch_types=[
        pltpu.VMEM((_EDGE_PAD,), jnp.float32),
        pltpu.VMEM((_PW,), jnp.float32),
        pltpu.VMEM((_PW,), jnp.int32),
        pltpu.VMEM((_CH, _HID), jnp.float32),
        pltpu.SemaphoreType.DMA,
    ],
    compiler_params=pltpu.CompilerParams(
        needs_layout_passes=False, use_tc_tiling_on_sc=False),
)(_sc_body)


def kernel(ts_values, table, bin_edges):
    table_n = _normalize_table(table)
    edges = jnp.full((_EDGE_PAD,), jnp.inf, dtype=jnp.float32)
    edges = edges.at[: _VOCAB + 1].set(bin_edges)
    emb, toks = _sc_lookup(ts_values, table_n, edges)
    return (emb, toks)


# speedup vs baseline: 112.0968x; 1.3030x over previous
import functools

import jax
import jax.numpy as jnp
from jax import lax
from jax.experimental import pallas as pl
from jax.experimental.pallas import tpu as pltpu
from jax.experimental.pallas import tpu_sc as plsc

_VOCAB = 1024
_HID = 64
_N = 1048576

_EDGE_PAD = 2048

_info = plsc.get_sparse_core_info()
_NC, _NS, _L = _info.num_cores, _info.num_subcores, _info.num_lanes
_NW = _NC * _NS
_PW = _N // _NW
_CH = 128
_NCHUNK = _PW // _CH


def _normalize_body(t_ref, o_ref):
    t = t_ref[...]
    ss = jnp.sum(t * t, axis=1, keepdims=True)
    norm = jnp.sqrt(ss)
    scale = jnp.where(norm > 1.0, 1.0 / norm, jnp.ones_like(norm))
    o_ref[...] = t * scale


def _normalize_table(table):
    return pl.pallas_call(
        _normalize_body,
        out_shape=jax.ShapeDtypeStruct((_VOCAB, _HID), jnp.float32),
    )(table)


_NBUF = 4


def _sc_body(vals_hbm, table_hbm, edges_hbm, emb_out, tok_out,
             edges_v, vals_v, toks_v, rows0, rows1, rows2, rows3,
             gsem0, gsem1, gsem2, gsem3, wsem0, wsem1, wsem2, wsem3):
    rows = (rows0, rows1, rows2, rows3)
    gsem = (gsem0, gsem1, gsem2, gsem3)
    wsem = (wsem0, wsem1, wsem2, wsem3)

    wid = lax.axis_index("s") * _NC + lax.axis_index("c")
    base = wid * _PW

    pltpu.sync_copy(edges_hbm, edges_v)
    pltpu.sync_copy(vals_hbm.at[pl.ds(base, _PW)], vals_v)

    def compute_tokens(c):
        for j in range(_CH // _L):
            v = vals_v[pl.ds(c * _CH + j * _L, _L)]
            pos = jnp.zeros((_L,), jnp.int32)
            k = _VOCAB
            while k >= 1:
                e = plsc.load_gather(edges_v, [pos + (k - 1)])
                pos = jnp.where(e < v, pos + k, pos)
                k //= 2
            tok = jnp.clip(pos - 1, 0, _VOCAB - 1)
            toks_v[pl.ds(c * _CH + j * _L, _L)] = tok

    def g_desc(c, b):
        idx = toks_v.at[pl.ds(c * _CH, _CH)]
        return pltpu.make_async_copy(table_hbm.at[idx], rows[b], gsem[b])

    def wb_desc(c, b):
        return pltpu.make_async_copy(
            rows[b], emb_out.at[pl.ds(base + c * _CH, _CH)], wsem[b])

    for b in range(_NBUF):
        compute_tokens(b)
        g_desc(b, b).start()

    def body(i, carry):
        for b in range(_NBUF):
            c = i * _NBUF + b
            g_desc(c, b).wait()
            wb_desc(c, b).start()

            @pl.when(i < _NCHUNK // _NBUF - 1)
            def _():
                compute_tokens(c + _NBUF)
                wb_desc(c, b).wait()
                g_desc(c + _NBUF, b).start()

        return carry

    lax.fori_loop(0, _NCHUNK // _NBUF, body, 0)

    for b in range(_NBUF):
        wb_desc(_NCHUNK - _NBUF + b, b).wait()

    pltpu.sync_copy(toks_v, tok_out.at[pl.ds(base, _PW)])


_sc_lookup = functools.partial(
    pl.kernel,
    mesh=plsc.VectorSubcoreMesh(core_axis_name="c", subcore_axis_name="s"),
    out_type=[
        jax.ShapeDtypeStruct((_N, _HID), jnp.float32),
        jax.ShapeDtypeStruct((_N,), jnp.int32),
    ],
    scratch_types=[
        pltpu.VMEM((_EDGE_PAD,), jnp.float32),
        pltpu.VMEM((_PW,), jnp.float32),
        pltpu.VMEM((_PW,), jnp.int32),
    ] + [pltpu.VMEM((_CH, _HID), jnp.float32) for _ in range(_NBUF)]
      + [pltpu.SemaphoreType.DMA for _ in range(2 * _NBUF)],
    compiler_params=pltpu.CompilerParams(
        needs_layout_passes=False, use_tc_tiling_on_sc=False),
)(_sc_body)


def kernel(ts_values, table, bin_edges):
    table_n = _normalize_table(table)
    edges = jnp.full((_EDGE_PAD,), jnp.inf, dtype=jnp.float32)
    edges = edges.at[: _VOCAB + 1].set(bin_edges)
    emb, toks = _sc_lookup(ts_values, table_n, edges)
    return (emb, toks)
